# baseline (speedup 1.0000x reference)
import functools

import jax
import jax.numpy as jnp
from jax import lax
from jax.experimental import pallas as pl
from jax.experimental.pallas import tpu as pltpu
from jax.experimental.pallas import tpu_sc as plsc

N_NODES = 10000
FDIM = 128
N_CORES = 2
N_SUBCORES = 16
N_TILES = N_CORES * N_SUBCORES
N_PAD = 10240
ROWS_PER_TILE = N_PAD // N_SUBCORES
CH = 128
CHUNKS_PER_TILE = 80
E_REAL = 320000
REAL_ROWS = E_REAL // CH
E_PAD = N_TILES * CHUNKS_PER_TILE * CH
PAD_ROWS = E_PAD // CH - REAL_ROWS
NBUF = 2
NIBUF = 4


CHB = 1000
HCHUNKS = E_REAL // CHB // N_TILES


def _sc_degree(dst_big, ones_ch, zvec):
    mesh = plsc.VectorSubcoreMesh(core_axis_name="c", subcore_axis_name="s")

    @functools.partial(
        pl.kernel,
        out_type=jax.ShapeDtypeStruct((N_CORES, N_PAD), jnp.float32),
        mesh=mesh,
        scratch_types=(
            [pltpu.VMEM_SHARED((N_PAD,), jnp.float32)]
            + [pltpu.VMEM((1, CHB), jnp.int32) for _ in range(HCHUNKS)]
            + [pltpu.VMEM((CHB,), jnp.float32)]
            + [pltpu.SemaphoreType.DMA for _ in range(HCHUNKS)]
        ),
    )
    def deg_kernel(dst_hbm, ones_hbm, zvec_hbm, deg_out, deg_sh, *rest):
        di = rest[:HCHUNKS]
        ones_v = rest[HCHUNKS]
        sems = rest[HCHUNKS + 1:]
        cid = lax.axis_index("c")
        sid = lax.axis_index("s")
        wid = cid * N_SUBCORES + sid
        rbase = pl.multiple_of(sid * ROWS_PER_TILE, 8)
        for k in range(HCHUNKS):
            pltpu.make_async_copy(dst_hbm.at[wid * HCHUNKS + k],
                                  di[k].at[0], sems[k]).start()
        pltpu.sync_copy(zvec_hbm, deg_sh.at[pl.ds(rbase, ROWS_PER_TILE)])
        pltpu.sync_copy(ones_hbm, ones_v)
        plsc.subcore_barrier()
        for k in range(HCHUNKS):
            pltpu.make_async_copy(dst_hbm.at[0], di[k].at[0], sems[k]).wait()
            pltpu.sync_copy(ones_v, deg_sh.at[di[k].at[0]], add=True)

        plsc.subcore_barrier()
        pltpu.sync_copy(deg_sh.at[pl.ds(rbase, ROWS_PER_TILE)],
                        deg_out.at[cid, pl.ds(rbase, ROWS_PER_TILE)])

    return deg_kernel(dst_big, ones_ch, zvec)


def _sc_aggregate(src_rows, dst_rows, psrc_rows, pdst_rows, g, dinv_pad,
                  zrow, zvec):
    mesh = plsc.VectorSubcoreMesh(core_axis_name="c", subcore_axis_name="s")

    @functools.partial(
        pl.kernel,
        out_type=(jax.ShapeDtypeStruct((N_CORES, N_PAD, FDIM), jnp.float32),
                  jax.ShapeDtypeStruct((N_CORES, N_PAD), jnp.float32)),
        mesh=mesh,
        scratch_types=(
            [pltpu.VMEM_SHARED((N_PAD, FDIM), jnp.float32),
             pltpu.VMEM_SHARED((N_PAD,), jnp.float32),
             pltpu.VMEM_SHARED((N_PAD,), jnp.float32)]
            + [pltpu.VMEM((1, CH), jnp.int32) for _ in range(2 * NIBUF)]
            + [pltpu.VMEM((CH, FDIM), jnp.float32) for _ in range(NBUF)]
            + [pltpu.VMEM((CH,), jnp.float32) for _ in range(NBUF)]
            + [pltpu.SemaphoreType.DMA for _ in range(2 * NIBUF + 2 * NBUF)]
        ),
    )
    def agg_kernel(src_hbm, dst_hbm, psrc_hbm, pdst_hbm, g_hbm, dinv_hbm,
                   zrow_hbm, zvec_hbm, acc_out, c_out, acc_sh, c_sh,
                   dinv_sh, *bufs):
        sb = bufs[:NIBUF]
        db = bufs[NIBUF:2 * NIBUF]
        k = 2 * NIBUF
        rows = bufs[k:k + NBUF]
        vals = bufs[k + NBUF:k + 2 * NBUF]
        k += 2 * NBUF
        sem_s = bufs[k:k + NIBUF]
        sem_di = bufs[k + NIBUF:k + 2 * NIBUF]
        k += 2 * NIBUF
        sem_g = bufs[k:k + NBUF]
        sem_d = bufs[k + NBUF:k + 2 * NBUF]
        cid = lax.axis_index("c")
        sid = lax.axis_index("s")
        wid = cid * N_SUBCORES + sid
        rbase = pl.multiple_of(sid * ROWS_PER_TILE, 8)

        def fetch_idx(ch, j):
            r = wid * CHUNKS_PER_TILE + ch

            @pl.when(r < REAL_ROWS)
            def _():
                pltpu.make_async_copy(src_hbm.at[r], sb[j].at[0],
                                      sem_s[j]).start()
                pltpu.make_async_copy(dst_hbm.at[r], db[j].at[0],
                                      sem_di[j]).start()

            @pl.when(r >= REAL_ROWS)
            def _():
                pltpu.make_async_copy(psrc_hbm.at[r - REAL_ROWS], sb[j].at[0],
                                      sem_s[j]).start()
                pltpu.make_async_copy(pdst_hbm.at[r - REAL_ROWS], db[j].at[0],
                                      sem_di[j]).start()

        def wait_idx(j):
            pltpu.make_async_copy(src_hbm.at[0], sb[j].at[0], sem_s[j]).wait()
            pltpu.make_async_copy(dst_hbm.at[0], db[j].at[0], sem_di[j]).wait()

        def fetch_data(j, b):
            pltpu.make_async_copy(g_hbm.at[sb[j].at[0]], rows[b],
                                  sem_g[b]).start()
            pltpu.make_async_copy(dinv_sh.at[db[j].at[0]], vals[b],
                                  sem_d[b]).start()

        def consume(j, b):
            pltpu.make_async_copy(g_hbm.at[sb[j].at[0]], rows[b],
                                  sem_g[b]).wait()
            pltpu.sync_copy(rows[b], acc_sh.at[db[j].at[0]], add=True)
            pltpu.make_async_copy(dinv_sh.at[db[j].at[0]], vals[b],
                                  sem_d[b]).wait()
            pltpu.sync_copy(vals[b], c_sh.at[sb[j].at[0]], add=True)

        for j in range(NIBUF):
            fetch_idx(j, j)
        pltpu.sync_copy(zrow_hbm, acc_sh.at[pl.ds(rbase, ROWS_PER_TILE)])
        pltpu.sync_copy(zvec_hbm, c_sh.at[pl.ds(rbase, ROWS_PER_TILE)])
        pltpu.sync_copy(dinv_hbm.at[pl.ds(rbase, ROWS_PER_TILE)],
                        dinv_sh.at[pl.ds(rbase, ROWS_PER_TILE)])
        plsc.subcore_barrier()
        for b in range(NBUF):
            wait_idx(b)
            fetch_data(b, b)

        @pl.loop(0, CHUNKS_PER_TILE - NIBUF, step=NIBUF)
        def _(i):
            for b in range(NIBUF):
                j = b
                b2 = b % NBUF
                consume(j, b2)
                fetch_idx(i + b + NIBUF, j)
                jn = (b + NBUF) % NIBUF
                wait_idx(jn)
                fetch_data(jn, b2)

        for b in range(NIBUF):
            ch = CHUNKS_PER_TILE - NIBUF + b
            consume(b, b % NBUF)
            if ch + NBUF < CHUNKS_PER_TILE:
                jn = (b + NBUF) % NIBUF
                wait_idx(jn)
                fetch_data(jn, b % NBUF)

        plsc.subcore_barrier()
        pltpu.sync_copy(acc_sh.at[pl.ds(rbase, ROWS_PER_TILE)],
                        acc_out.at[cid, pl.ds(rbase, ROWS_PER_TILE)])
        pltpu.sync_copy(c_sh.at[pl.ds(rbase, ROWS_PER_TILE)],
                        c_out.at[cid, pl.ds(rbase, ROWS_PER_TILE)])

    return agg_kernel(src_rows, dst_rows, psrc_rows, pdst_rows, g, dinv_pad,
                      zrow, zvec)


def _tc_matmul(x, W):
    BLK = 2000

    def body(x_ref, w_ref, o_ref):
        o_ref[...] = jnp.dot(x_ref[...], w_ref[...],
                             preferred_element_type=jnp.float32,
                             precision=lax.Precision.HIGHEST)

    return pl.pallas_call(
        body,
        grid=(N_NODES // BLK,),
        in_specs=[pl.BlockSpec((BLK, FDIM), lambda i: (i, 0)),
                  pl.BlockSpec((FDIM, FDIM), lambda i: (0, 0))],
        out_specs=pl.BlockSpec((BLK, FDIM), lambda i: (i, 0)),
        out_shape=jax.ShapeDtypeStruct((N_NODES, FDIM), jnp.float32),
    )(x, W)


def _tc_scale(h, degp):
    BLK = 2000

    def body(h_ref, deg_ref, g_ref, dinv_ref):
        deg = deg_ref[0] + deg_ref[1] + 1.0
        dinv = lax.rsqrt(deg)
        g_ref[...] = h_ref[...] * dinv
        dinv_ref[...] = dinv

    return pl.pallas_call(
        body,
        grid=(N_NODES // BLK,),
        in_specs=[pl.BlockSpec((BLK, FDIM), lambda i: (i, 0)),
                  pl.BlockSpec((2, BLK, 1), lambda i: (0, i, 0))],
        out_specs=[pl.BlockSpec((BLK, FDIM), lambda i: (i, 0)),
                   pl.BlockSpec((BLK, 1), lambda i: (i, 0))],
        out_shape=[jax.ShapeDtypeStruct((N_NODES, FDIM), jnp.float32),
                   jax.ShapeDtypeStruct((N_NODES, 1), jnp.float32)],
    )(h, degp.reshape(N_CORES, N_PAD, 1))


def _tc_final(accp, g, dinv2, cp, b1r, W2, b2r):
    BLK = 2000
    nb = N_NODES // BLK

    def body(acc_ref, g_ref, dinv_ref, c_ref, b1_ref, w2_ref, b2_ref,
             o_ref, v_ref):
        i = pl.program_id(0)

        @pl.when(i == 0)
        def _():
            v_ref[...] = jnp.zeros_like(v_ref)

        dinv = dinv_ref[...]
        agg = acc_ref[0] + acc_ref[1]
        h1 = jnp.maximum(dinv * (agg + g_ref[...]) + b1_ref[...], 0.0)
        w = dinv * (c_ref[0] + c_ref[1] + dinv)
        v_ref[...] += jnp.sum(w * h1, axis=0, keepdims=True)

        @pl.when(i == nb - 1)
        def _():
            o_ref[...] = (jnp.dot(v_ref[...], w2_ref[...],
                                  preferred_element_type=jnp.float32,
                                  precision=lax.Precision.HIGHEST)
                          * (1.0 / N_NODES) + b2_ref[...])

    return pl.pallas_call(
        body,
        grid=(nb,),
        in_specs=[pl.BlockSpec((2, BLK, FDIM), lambda i: (0, i, 0)),
                  pl.BlockSpec((BLK, FDIM), lambda i: (i, 0)),
                  pl.BlockSpec((BLK, 1), lambda i: (i, 0)),
                  pl.BlockSpec((2, BLK, 1), lambda i: (0, i, 0)),
                  pl.BlockSpec((1, FDIM), lambda i: (0, 0)),
                  pl.BlockSpec((FDIM, FDIM), lambda i: (0, 0)),
                  pl.BlockSpec((1, FDIM), lambda i: (0, 0))],
        out_specs=pl.BlockSpec((1, FDIM), lambda i: (0, 0)),
        out_shape=jax.ShapeDtypeStruct((1, FDIM), jnp.float32),
        scratch_shapes=[pltpu.VMEM((1, FDIM), jnp.float32)],
    )(accp, g, dinv2, cp.reshape(N_CORES, N_PAD, 1), b1r, W2, b2r)


def kernel(x, edge_index, W1, b1, W2, b2):
    src = edge_index[0].astype(jnp.int32)
    dst = edge_index[1].astype(jnp.int32)
    src_rows = src.reshape(REAL_ROWS, CH)
    dst_rows = dst.reshape(REAL_ROWS, CH)
    dst_big = dst.reshape(E_REAL // CHB, CHB)
    n_pad_e = PAD_ROWS * CH
    psrc_rows = ((jnp.arange(n_pad_e, dtype=jnp.int32) * 131)
                 % N_NODES).reshape(PAD_ROWS, CH)
    pdst_rows = (N_NODES + jnp.arange(n_pad_e, dtype=jnp.int32)
                 % (N_PAD - N_NODES)).reshape(PAD_ROWS, CH)
    ones_ch = jnp.ones((CHB,), jnp.float32)
    zvec = jnp.zeros((ROWS_PER_TILE,), jnp.float32)
    zrow = jnp.zeros((ROWS_PER_TILE, FDIM), jnp.float32)

    degp = _sc_degree(dst_big, ones_ch, zvec)
    h = _tc_matmul(x, W1)
    g, dinv2 = _tc_scale(h, degp)
    dinv_pad = jnp.concatenate(
        [dinv2.reshape(N_NODES),
         jnp.zeros((N_PAD - N_NODES,), jnp.float32)])
    accp, cp = _sc_aggregate(src_rows, dst_rows, psrc_rows, pdst_rows,
                             g, dinv_pad, zrow, zvec)
    return _tc_final(accp, g, dinv2, cp,
                     b1.reshape(1, FDIM), W2, b2.reshape(1, FDIM))


# speedup vs baseline: 1.0182x; 1.0182x over previous
import functools

import jax
import jax.numpy as jnp
from jax import lax
from jax.experimental import pallas as pl
from jax.experimental.pallas import tpu as pltpu
from jax.experimental.pallas import tpu_sc as plsc

N_NODES = 10000
FDIM = 128
N_CORES = 2
N_SUBCORES = 16
N_TILES = N_CORES * N_SUBCORES
N_PAD = 10240
ROWS_PER_TILE = N_PAD // N_SUBCORES
CH = 128
CHUNKS_PER_TILE = 80
E_REAL = 320000
REAL_ROWS = E_REAL // CH
E_PAD = N_TILES * CHUNKS_PER_TILE * CH
PAD_ROWS = E_PAD // CH - REAL_ROWS
NBUF = 2
NIBUF = 4


CHB = 1000
HCHUNKS = E_REAL // CHB // N_TILES


def _sc_degree(dst_big, ones_ch, zvec):
    mesh = plsc.VectorSubcoreMesh(core_axis_name="c", subcore_axis_name="s")

    @functools.partial(
        pl.kernel,
        out_type=jax.ShapeDtypeStruct((N_CORES, N_PAD), jnp.float32),
        mesh=mesh,
        scratch_types=(
            [pltpu.VMEM_SHARED((N_PAD,), jnp.float32)]
            + [pltpu.VMEM((1, CHB), jnp.int32) for _ in range(HCHUNKS)]
            + [pltpu.VMEM((CHB,), jnp.float32)]
            + [pltpu.SemaphoreType.DMA for _ in range(HCHUNKS)]
        ),
    )
    def deg_kernel(dst_hbm, ones_hbm, zvec_hbm, deg_out, deg_sh, *rest):
        di = rest[:HCHUNKS]
        ones_v = rest[HCHUNKS]
        sems = rest[HCHUNKS + 1:]
        cid = lax.axis_index("c")
        sid = lax.axis_index("s")
        wid = cid * N_SUBCORES + sid
        rbase = pl.multiple_of(sid * ROWS_PER_TILE, 8)
        for k in range(HCHUNKS):
            pltpu.make_async_copy(dst_hbm.at[wid * HCHUNKS + k],
                                  di[k].at[0], sems[k]).start()
        pltpu.sync_copy(zvec_hbm, deg_sh.at[pl.ds(rbase, ROWS_PER_TILE)])
        pltpu.sync_copy(ones_hbm, ones_v)
        plsc.subcore_barrier()
        for k in range(HCHUNKS):
            pltpu.make_async_copy(dst_hbm.at[0], di[k].at[0], sems[k]).wait()
            pltpu.sync_copy(ones_v, deg_sh.at[di[k].at[0]], add=True)

        plsc.subcore_barrier()
        pltpu.sync_copy(deg_sh.at[pl.ds(rbase, ROWS_PER_TILE)],
                        deg_out.at[cid, pl.ds(rbase, ROWS_PER_TILE)])

    return deg_kernel(dst_big, ones_ch, zvec)


def _sc_aggregate(src_rows, dst_rows, g, dinv_pad, zrow, zvec):
    mesh = plsc.VectorSubcoreMesh(core_axis_name="c", subcore_axis_name="s")

    @functools.partial(
        pl.kernel,
        out_type=(jax.ShapeDtypeStruct((N_CORES, N_PAD, FDIM), jnp.float32),
                  jax.ShapeDtypeStruct((N_CORES, N_PAD), jnp.float32)),
        mesh=mesh,
        scratch_types=(
            [pltpu.VMEM_SHARED((N_PAD, FDIM), jnp.float32),
             pltpu.VMEM_SHARED((N_PAD,), jnp.float32),
             pltpu.VMEM_SHARED((N_PAD,), jnp.float32)]
            + [pltpu.VMEM((1, CH), jnp.int32) for _ in range(2 * NIBUF)]
            + [pltpu.VMEM((CH, FDIM), jnp.float32) for _ in range(NBUF)]
            + [pltpu.VMEM((CH,), jnp.float32) for _ in range(NBUF)]
            + [pltpu.SemaphoreType.DMA for _ in range(2 * NIBUF + 2 * NBUF)]
        ),
    )
    def agg_kernel(src_hbm, dst_hbm, g_hbm, dinv_hbm, zrow_hbm, zvec_hbm,
                   acc_out, c_out, acc_sh, c_sh, dinv_sh, *bufs):
        sb = bufs[:NIBUF]
        db = bufs[NIBUF:2 * NIBUF]
        k = 2 * NIBUF
        rows = bufs[k:k + NBUF]
        vals = bufs[k + NBUF:k + 2 * NBUF]
        k += 2 * NBUF
        sem_s = bufs[k:k + NIBUF]
        sem_di = bufs[k + NIBUF:k + 2 * NIBUF]
        k += 2 * NIBUF
        sem_g = bufs[k:k + NBUF]
        sem_d = bufs[k + NBUF:k + 2 * NBUF]
        cid = lax.axis_index("c")
        sid = lax.axis_index("s")
        wid = cid * N_SUBCORES + sid
        rbase = pl.multiple_of(sid * ROWS_PER_TILE, 8)

        def fetch_idx(ch, j):
            r = wid * CHUNKS_PER_TILE + ch
            pltpu.make_async_copy(src_hbm.at[r], sb[j].at[0], sem_s[j]).start()
            pltpu.make_async_copy(dst_hbm.at[r], db[j].at[0], sem_di[j]).start()

        def wait_idx(j):
            pltpu.make_async_copy(src_hbm.at[0], sb[j].at[0], sem_s[j]).wait()
            pltpu.make_async_copy(dst_hbm.at[0], db[j].at[0], sem_di[j]).wait()

        def fetch_data(j, b):
            pltpu.make_async_copy(g_hbm.at[sb[j].at[0]], rows[b],
                                  sem_g[b]).start()
            pltpu.make_async_copy(dinv_sh.at[db[j].at[0]], vals[b],
                                  sem_d[b]).start()

        def consume(j, b):
            pltpu.make_async_copy(g_hbm.at[sb[j].at[0]], rows[b],
                                  sem_g[b]).wait()
            pltpu.sync_copy(rows[b], acc_sh.at[db[j].at[0]], add=True)
            pltpu.make_async_copy(dinv_sh.at[db[j].at[0]], vals[b],
                                  sem_d[b]).wait()
            pltpu.sync_copy(vals[b], c_sh.at[sb[j].at[0]], add=True)

        for j in range(NIBUF):
            fetch_idx(j, j)
        pltpu.sync_copy(zrow_hbm, acc_sh.at[pl.ds(rbase, ROWS_PER_TILE)])
        pltpu.sync_copy(zvec_hbm, c_sh.at[pl.ds(rbase, ROWS_PER_TILE)])
        pltpu.sync_copy(dinv_hbm.at[pl.ds(rbase, ROWS_PER_TILE)],
                        dinv_sh.at[pl.ds(rbase, ROWS_PER_TILE)])
        plsc.subcore_barrier()
        for b in range(NBUF):
            wait_idx(b)
            fetch_data(b, b)

        @pl.loop(0, CHUNKS_PER_TILE - NIBUF, step=NIBUF)
        def _(i):
            for b in range(NIBUF):
                j = b
                b2 = b % NBUF
                consume(j, b2)
                fetch_idx(i + b + NIBUF, j)
                jn = (b + NBUF) % NIBUF
                wait_idx(jn)
                fetch_data(jn, b2)

        for b in range(NIBUF):
            ch = CHUNKS_PER_TILE - NIBUF + b
            consume(b, b % NBUF)
            if ch + NBUF < CHUNKS_PER_TILE:
                jn = (b + NBUF) % NIBUF
                wait_idx(jn)
                fetch_data(jn, b % NBUF)

        plsc.subcore_barrier()
        pltpu.sync_copy(acc_sh.at[pl.ds(rbase, ROWS_PER_TILE)],
                        acc_out.at[cid, pl.ds(rbase, ROWS_PER_TILE)])
        pltpu.sync_copy(c_sh.at[pl.ds(rbase, ROWS_PER_TILE)],
                        c_out.at[cid, pl.ds(rbase, ROWS_PER_TILE)])

    return agg_kernel(src_rows, dst_rows, g, dinv_pad, zrow, zvec)


def _tc_matmul(x, W):
    BLK = 2000

    def body(x_ref, w_ref, o_ref):
        o_ref[...] = jnp.dot(x_ref[...], w_ref[...],
                             preferred_element_type=jnp.float32,
                             precision=lax.Precision.HIGHEST)

    return pl.pallas_call(
        body,
        grid=(N_NODES // BLK,),
        in_specs=[pl.BlockSpec((BLK, FDIM), lambda i: (i, 0)),
                  pl.BlockSpec((FDIM, FDIM), lambda i: (0, 0))],
        out_specs=pl.BlockSpec((BLK, FDIM), lambda i: (i, 0)),
        out_shape=jax.ShapeDtypeStruct((N_NODES, FDIM), jnp.float32),
    )(x, W)


def _tc_scale(h, degp):
    BLK = 2000

    def body(h_ref, deg_ref, g_ref, dinv_ref):
        deg = deg_ref[0] + deg_ref[1] + 1.0
        dinv = lax.rsqrt(deg)
        g_ref[...] = h_ref[...] * dinv
        dinv_ref[...] = dinv

    return pl.pallas_call(
        body,
        grid=(N_NODES // BLK,),
        in_specs=[pl.BlockSpec((BLK, FDIM), lambda i: (i, 0)),
                  pl.BlockSpec((2, BLK, 1), lambda i: (0, i, 0))],
        out_specs=[pl.BlockSpec((BLK, FDIM), lambda i: (i, 0)),
                   pl.BlockSpec((BLK, 1), lambda i: (i, 0))],
        out_shape=[jax.ShapeDtypeStruct((N_NODES, FDIM), jnp.float32),
                   jax.ShapeDtypeStruct((N_NODES, 1), jnp.float32)],
    )(h, degp.reshape(N_CORES, N_PAD, 1))


def _tc_final(accp, g, dinv2, cp, b1r, W2, b2r):
    BLK = 2000
    nb = N_NODES // BLK

    def body(acc_ref, g_ref, dinv_ref, c_ref, b1_ref, w2_ref, b2_ref,
             o_ref, v_ref):
        i = pl.program_id(0)

        @pl.when(i == 0)
        def _():
            v_ref[...] = jnp.zeros_like(v_ref)

        dinv = dinv_ref[...]
        agg = acc_ref[0] + acc_ref[1]
        h1 = jnp.maximum(dinv * (agg + g_ref[...]) + b1_ref[...], 0.0)
        w = dinv * (c_ref[0] + c_ref[1] + dinv)
        v_ref[...] += jnp.sum(w * h1, axis=0, keepdims=True)

        @pl.when(i == nb - 1)
        def _():
            o_ref[...] = (jnp.dot(v_ref[...], w2_ref[...],
                                  preferred_element_type=jnp.float32,
                                  precision=lax.Precision.HIGHEST)
                          * (1.0 / N_NODES) + b2_ref[...])

    return pl.pallas_call(
        body,
        grid=(nb,),
        in_specs=[pl.BlockSpec((2, BLK, FDIM), lambda i: (0, i, 0)),
                  pl.BlockSpec((BLK, FDIM), lambda i: (i, 0)),
                  pl.BlockSpec((BLK, 1), lambda i: (i, 0)),
                  pl.BlockSpec((2, BLK, 1), lambda i: (0, i, 0)),
                  pl.BlockSpec((1, FDIM), lambda i: (0, 0)),
                  pl.BlockSpec((FDIM, FDIM), lambda i: (0, 0)),
                  pl.BlockSpec((1, FDIM), lambda i: (0, 0))],
        out_specs=pl.BlockSpec((1, FDIM), lambda i: (0, 0)),
        out_shape=jax.ShapeDtypeStruct((1, FDIM), jnp.float32),
        scratch_shapes=[pltpu.VMEM((1, FDIM), jnp.float32)],
    )(accp, g, dinv2, cp.reshape(N_CORES, N_PAD, 1), b1r, W2, b2r)


def kernel(x, edge_index, W1, b1, W2, b2):
    src = edge_index[0].astype(jnp.int32)
    dst = edge_index[1].astype(jnp.int32)
    dst_big = dst.reshape(E_REAL // CHB, CHB)
    n_pad_e = PAD_ROWS * CH
    pad_src = (jnp.arange(n_pad_e, dtype=jnp.int32) * 131) % N_NODES
    pad_dst = N_NODES + (jnp.arange(n_pad_e, dtype=jnp.int32)
                         % (N_PAD - N_NODES))
    src_rows = jnp.concatenate([src, pad_src]).reshape(E_PAD // CH, CH)
    dst_rows = jnp.concatenate([dst, pad_dst]).reshape(E_PAD // CH, CH)
    ones_ch = jnp.ones((CHB,), jnp.float32)
    zvec = jnp.zeros((ROWS_PER_TILE,), jnp.float32)
    zrow = jnp.zeros((ROWS_PER_TILE, FDIM), jnp.float32)

    degp = _sc_degree(dst_big, ones_ch, zvec)
    h = _tc_matmul(x, W1)
    g, dinv2 = _tc_scale(h, degp)
    dinv_pad = jnp.concatenate(
        [dinv2.reshape(N_NODES),
         jnp.zeros((N_PAD - N_NODES,), jnp.float32)])
    accp, cp = _sc_aggregate(src_rows, dst_rows, g, dinv_pad, zrow, zvec)
    return _tc_final(accp, g, dinv2, cp,
                     b1.reshape(1, FDIM), W2, b2.reshape(1, FDIM))
